# baseline (speedup 1.0000x reference)
import functools
import jax
import jax.numpy as jnp
from jax import lax
from jax.experimental import pallas as pl
from jax.experimental.pallas import tpu as pltpu
from jax.experimental.pallas import tpu_sc as plsc

_D = 64
_NC = 2
_NS = 16
_NW = _NC * _NS
_LANES = 16
_BB = 128


def _transpose_scale(table_t, V):
  vb = 4096
  grid = pl.cdiv(V, vb)

  def body(i_ref, o_ref):
    r = lax.broadcasted_iota(jnp.int32, (_D, _D), 0)
    c = lax.broadcasted_iota(jnp.int32, (_D, _D), 1)
    diag8 = jnp.where(r == c, 8.0, 0.0).astype(jnp.float32)
    o_ref[...] = lax.dot_general(
        i_ref[...], diag8, (((0,), (0,)), ((), ())),
        precision=lax.Precision.HIGHEST,
        preferred_element_type=jnp.float32)

  return pl.pallas_call(
      body,
      grid=(grid,),
      in_specs=[pl.BlockSpec((_D, vb), lambda i: (0, i))],
      out_specs=pl.BlockSpec((vb, _D), lambda i: (i, 0)),
      out_shape=jax.ShapeDtypeStruct((V, _D), jnp.float32),
  )(table_t)


def _emb_lookup(x_t, table, B, T):
  assert B == _NW * _BB

  mesh = plsc.VectorSubcoreMesh(
      core_axis_name="c", subcore_axis_name="s",
      num_cores=_NC, num_subcores=_NS)

  @functools.partial(
      pl.kernel,
      out_type=jax.ShapeDtypeStruct((T, _D // 8, B // _BB, 8, _BB),
                                    jnp.float32),
      mesh=mesh,
      compiler_params=pltpu.CompilerParams(use_tc_tiling_on_sc=False,
                                           needs_layout_passes=False),
      scratch_types=[
          pltpu.VMEM((T, _BB), jnp.int32),
          pltpu.VMEM((2, _BB, _D), jnp.float32),
          pltpu.VMEM((2, 8, 8, _BB + 1), jnp.float32),
          pltpu.SemaphoreType.DMA,
          pltpu.SemaphoreType.DMA,
          pltpu.SemaphoreType.DMA,
          pltpu.SemaphoreType.DMA,
      ],
  )
  def k(xt_hbm, table_hbm, out_hbm, idx_v, rows_v, tiles_v,
        sem0, sem1, osem0, osem1):
    wid = lax.axis_index("s") * _NC + lax.axis_index("c")
    b0 = pl.multiple_of(wid * _BB, _BB)
    pltpu.sync_copy(xt_hbm.at[:, pl.ds(b0, _BB)], idx_v)

    sems = (sem0, sem1)

    def start_gather(t, b):
      pltpu.async_copy(table_hbm.at[idx_v.at[t]], rows_v.at[b], sems[b])

    start_gather(0, 0)
    start_gather(1, 1)

    iota = lax.iota(jnp.int32, _LANES)
    js_g = [lax.shift_right_logical(iota, 3) + 2 * g for g in range(4)]
    jr = lax.bitwise_and(iota, 7)
    osems = (osem0, osem1)

    def pair_body(p, _):
      for b in range(2):
        t = p * 2 + b
        buf = rows_v.at[b]
        st = tiles_v.at[b]
        pltpu.make_async_copy(table_hbm.at[pl.ds(0, _BB)],
                              buf, sems[b]).wait()
        @pl.when(t >= 2)
        def _():
          pltpu.make_async_copy(out_hbm.at[0, :, 0],
                                st.at[:, :, pl.ds(0, _BB)],
                                osems[b]).wait()

        @plsc.parallel_loop(0, _BB, 1, unroll=8)
        def _permute(bl):
          bl_s = jnp.broadcast_to(bl, (_LANES,))
          vs = [buf[bl, pl.ds(16 * g, _LANES)] for g in range(4)]
          for g in range(4):
            plsc.store_scatter(st, [js_g[g], jr, bl_s], vs[g])

        pltpu.async_copy(st.at[:, :, pl.ds(0, _BB)],
                         out_hbm.at[t, :, wid], osems[b])

        @pl.when(t + 2 < T)
        def _():
          start_gather(t + 2, b)
      return ()

    lax.fori_loop(0, T // 2, pair_body, ())
    for b in range(2):
      pltpu.make_async_copy(out_hbm.at[0, :, 0],
                            tiles_v.at[b, :, :, pl.ds(0, _BB)],
                            osems[b]).wait()

  return k(x_t, table)


def kernel(x, emb_weight):
  b, t = x.shape
  v = emb_weight.shape[0]
  table8 = _transpose_scale(emb_weight.T, v)
  out5 = _emb_lookup(x.T, table8, b, t)
  out = out5.transpose(2, 4, 0, 1, 3).reshape(b, t, _D)
  return out


# speedup vs baseline: 1.1509x; 1.1509x over previous
import functools
import jax
import jax.numpy as jnp
from jax import lax
from jax.experimental import pallas as pl
from jax.experimental.pallas import tpu as pltpu
from jax.experimental.pallas import tpu_sc as plsc

_D = 64
_NC = 2
_NS = 16
_NW = _NC * _NS
_LANES = 16
_BB = 128


def _transpose_scale(table_t, V):
  vb = 4096
  grid = pl.cdiv(V, vb)

  def body(i_ref, o_ref):
    o_ref[...] = jnp.transpose(i_ref[...], (1, 0)) * 8.0

  return pl.pallas_call(
      body,
      grid=(grid,),
      in_specs=[pl.BlockSpec((_D, vb), lambda i: (0, i))],
      out_specs=pl.BlockSpec((vb, _D), lambda i: (i, 0)),
      out_shape=jax.ShapeDtypeStruct((V, _D), jnp.float32),
  )(table_t)


def _emb_lookup(x_t, table, B, T):
  assert B == _NW * _BB

  mesh = plsc.VectorSubcoreMesh(
      core_axis_name="c", subcore_axis_name="s",
      num_cores=_NC, num_subcores=_NS)

  @functools.partial(
      pl.kernel,
      out_type=jax.ShapeDtypeStruct((T, _D // 8, B // _BB, 8, _BB),
                                    jnp.float32),
      mesh=mesh,
      compiler_params=pltpu.CompilerParams(use_tc_tiling_on_sc=False,
                                           needs_layout_passes=False),
      scratch_types=[
          pltpu.VMEM((T, _BB), jnp.int32),
          pltpu.VMEM((2, _BB, _D), jnp.float32),
          pltpu.VMEM((2, 8, 8, _BB + 1), jnp.float32),
          pltpu.SemaphoreType.DMA,
          pltpu.SemaphoreType.DMA,
          pltpu.SemaphoreType.DMA,
          pltpu.SemaphoreType.DMA,
      ],
  )
  def k(xt_hbm, table_hbm, out_hbm, idx_v, rows_v, tiles_v,
        sem0, sem1, osem0, osem1):
    wid = lax.axis_index("s") * _NC + lax.axis_index("c")
    b0 = pl.multiple_of(wid * _BB, _BB)
    pltpu.sync_copy(xt_hbm.at[:, pl.ds(b0, _BB)], idx_v)

    sems = (sem0, sem1)

    def start_gather(t, b):
      pltpu.async_copy(table_hbm.at[idx_v.at[t]], rows_v.at[b], sems[b])

    start_gather(0, 0)
    start_gather(1, 1)

    iota = lax.iota(jnp.int32, _LANES)
    js_g = [lax.shift_right_logical(iota, 3) + 2 * g for g in range(4)]
    jr = lax.bitwise_and(iota, 7)
    osems = (osem0, osem1)

    def pair_body(p, _):
      for b in range(2):
        t = p * 2 + b
        buf = rows_v.at[b]
        st = tiles_v.at[b]
        pltpu.make_async_copy(table_hbm.at[pl.ds(0, _BB)],
                              buf, sems[b]).wait()
        @pl.when(t >= 2)
        def _():
          pltpu.make_async_copy(out_hbm.at[0, :, 0],
                                st.at[:, :, pl.ds(0, _BB)],
                                osems[b]).wait()

        @plsc.parallel_loop(0, _BB, 1, unroll=8)
        def _permute(bl):
          bl_s = jnp.broadcast_to(bl, (_LANES,))
          vs = [buf[bl, pl.ds(16 * g, _LANES)] for g in range(4)]
          for g in range(4):
            plsc.store_scatter(st, [js_g[g], jr, bl_s], vs[g])

        pltpu.async_copy(st.at[:, :, pl.ds(0, _BB)],
                         out_hbm.at[t, :, wid], osems[b])

        @pl.when(t + 2 < T)
        def _():
          start_gather(t + 2, b)
      return ()

    lax.fori_loop(0, T // 2, pair_body, ())
    for b in range(2):
      pltpu.make_async_copy(out_hbm.at[0, :, 0],
                            tiles_v.at[b, :, :, pl.ds(0, _BB)],
                            osems[b]).wait()

  return k(x_t, table)


def kernel(x, emb_weight):
  b, t = x.shape
  v = emb_weight.shape[0]
  table8 = _transpose_scale(emb_weight.T, v)
  out5 = _emb_lookup(x.T, table8, b, t)
  out = out5.transpose(2, 4, 0, 1, 3).reshape(b, t, _D)
  return out
